# baseline (speedup 1.0000x reference)
import numpy as np
import jax
import jax.numpy as jnp
from jax import lax
from jax.experimental import pallas as pl
from jax.experimental.pallas import tpu as pltpu

B = 16384
D = 768
K = 4
N = 256
KN = K * N
R = 128
NH = 256

_U32 = np.uint32


def _np_tf2x32(k0, k1, x0, x1):
    ks2 = _U32(k0) ^ _U32(k1) ^ _U32(0x1BD11BDA)
    x0 = _U32(x0 + k0)
    x1 = _U32(x1 + k1)
    keys = [(k1, ks2, 1), (ks2, k0, 2), (k0, k1, 3), (k1, ks2, 4), (ks2, k0, 5)]
    rots = [(13, 15, 26, 6), (17, 29, 16, 24), (13, 15, 26, 6),
            (17, 29, 16, 24), (13, 15, 26, 6)]
    for (ka, kb, i), rr in zip(keys, rots):
        for r in rr:
            x0 = _U32((int(x0) + int(x1)) & 0xFFFFFFFF)
            x1 = _U32(((int(x1) << r) | (int(x1) >> (32 - r))) & 0xFFFFFFFF)
            x1 = x1 ^ x0
        x0 = _U32((int(x0) + int(ka)) & 0xFFFFFFFF)
        x1 = _U32((int(x1) + int(kb) + i) & 0xFFFFFFFF)
    return int(x0), int(x1)


_KG0, _KG1 = _np_tf2x32(0, 1234, 0, 2)


def _i32(x):
    return jnp.int32(np.int32(np.uint32(x)))


def _rotl(v, r):
    return lax.shift_left(v, jnp.int32(r)) | lax.shift_right_logical(
        v, jnp.int32(32 - r))


def _tf(k0, k1, x0, x1):
    ks2 = k0 ^ k1 ^ _i32(0x1BD11BDA)
    x0 = x0 + k0
    x1 = x1 + k1
    keys = [(k1, ks2, 1), (ks2, k0, 2), (k0, k1, 3), (k1, ks2, 4), (ks2, k0, 5)]
    rots = [(13, 15, 26, 6), (17, 29, 16, 24), (13, 15, 26, 6),
            (17, 29, 16, 24), (13, 15, 26, 6)]
    for (ka, kb, i), rr in zip(keys, rots):
        for r in rr:
            x0 = x0 + x1
            x1 = _rotl(x1, r)
            x1 = x1 ^ x0
        x0 = x0 + ka
        x1 = x1 + kb + jnp.int32(i)
    return x0, x1


def _bits_to_unit(bits):
    f = lax.bitcast_convert_type(
        lax.shift_right_logical(bits, jnp.int32(9)) | _i32(0x3F800000),
        jnp.float32)
    return f - jnp.float32(1.0)


def _uniform01(k0, k1):
    o0, o1 = _tf(k0, k1, jnp.zeros_like(k0), jnp.zeros_like(k1))
    return jnp.maximum(jnp.float32(0.0), _bits_to_unit(o0 ^ o1))


def _erf_inv(x):
    w = -jnp.log1p(-x * x)
    lt = w < jnp.float32(5.0)
    w1 = w - jnp.float32(2.5)
    p = jnp.float32(2.81022636e-08)
    for c in (3.43273939e-07, -3.5233877e-06, -4.39150654e-06, 0.00021858087,
              -0.00125372503, -0.00417768164, 0.246640727, 1.50140941):
        p = jnp.float32(c) + p * w1
    w2 = jnp.sqrt(w) - jnp.float32(3.0)
    q = jnp.float32(-0.000200214257)
    for c in (0.000100950558, 0.00134934322, -0.00367342844, 0.00573950773,
              -0.0076224613, 0.00943887047, 1.00167406, 2.83297682):
        q = jnp.float32(c) + q * w2
    return jnp.where(lt, p, q) * x


def _normal(k0, k1):
    lo = jnp.float32(-0.99999994)
    hi = jnp.float32(1.0)
    o0, o1 = _tf(k0, k1, jnp.zeros_like(k0), jnp.zeros_like(k1))
    u = _bits_to_unit(o0 ^ o1)
    u = jnp.maximum(lo, u * (hi - lo) + lo)
    return jnp.float32(1.41421354) * _erf_inv(u)


def _softplus(x):
    return jnp.maximum(x, jnp.float32(0.0)) + jnp.log1p(jnp.exp(-jnp.abs(x)))


def _accept_continue(X, V, U, d):
    sq = jnp.float32(1.0) - jnp.float32(0.0331) * (X * X)
    rhs = jnp.float32(0.5) * X + d * ((jnp.float32(1.0) - V) + jnp.log(V))
    return (U >= sq) & (jnp.log(U) >= rhs)


def _sampler_body(lat_ref, wmu_ref, bmu_ref, wsig_ref, bsig_ref, wp_ref,
                  bp_ref, sgn_ref, kc_ref, sx_ref, xmu_ref, xsig_ref, xp_ref):
    lat = lat_ref[...].astype(jnp.bfloat16)
    hi = jax.lax.Precision.HIGHEST
    x_mu = jnp.dot(lat, wmu_ref[...].astype(jnp.bfloat16),
                   preferred_element_type=jnp.float32) + bmu_ref[...]
    xs_l = jnp.dot(lat, wsig_ref[...].astype(jnp.bfloat16),
                   preferred_element_type=jnp.float32) + bsig_ref[...]
    xp_l = jnp.dot(lat, wp_ref[...].astype(jnp.bfloat16),
                   preferred_element_type=jnp.float32) + bp_ref[...]

    x_sig = _softplus(xs_l) + jnp.float32(1e-08)
    x_sig = jnp.where(x_sig > jnp.float32(4.0), jnp.float32(4.0), x_sig)
    x_sig = jnp.where(x_sig < jnp.float32(0.001), jnp.float32(0.001), x_sig)
    x_p = _softplus(xp_l) + jnp.float32(1e-08) + jnp.float32(0.1)
    x_p = jnp.where(x_p > jnp.float32(10.0), jnp.float32(10.0), x_p)

    xmu_ref[...] = x_mu
    xsig_ref[...] = x_sig
    xp_ref[...] = x_p

    kc = kc_ref[...]
    cols = lax.broadcasted_iota(jnp.int32, (R, KN), 1)
    mask = (cols % K == kc).astype(jnp.float32)
    s_rows = lax.broadcasted_iota(jnp.int32, (KN, N), 0)
    s_cols = lax.broadcasted_iota(jnp.int32, (KN, N), 1)
    S = (s_rows // K == s_cols).astype(jnp.float32)
    x_mu = jnp.dot(x_mu * mask, S, precision=hi,
                   preferred_element_type=jnp.float32)
    x_sig = jnp.dot(x_sig * mask, S, precision=hi,
                    preferred_element_type=jnp.float32)
    x_p = jnp.dot(x_p * mask, S, precision=hi,
                  preferred_element_type=jnp.float32)

    sgn_all = sgn_ref[...].astype(jnp.float32)
    for h in range(N // NH):
        cs = slice(h * NH, (h + 1) * NH)
        sx_ref[:, cs] = _sample_half(x_mu[:, cs], x_sig[:, cs], x_p[:, cs],
                                     sgn_all[:, cs], kc, h)


def _sample_half(x_mu, x_sig, x_p, sgn, kc, h):
    a = jnp.float32(1.0) / x_p
    mask_ge1 = a >= jnp.float32(1.0)
    alpha = jnp.where(mask_ge1, a, a + jnp.float32(1.0))
    third = jnp.float32(np.float32(1.0 / 3.0))
    d = alpha - third
    c = third / jnp.sqrt(d)

    rows = lax.broadcasted_iota(jnp.int32, (R, NH), 0)
    cols_n = lax.broadcasted_iota(jnp.int32, (R, NH), 1) + h * NH
    elem = (pl.program_id(0) * R + rows) * KN + cols_n * K + kc
    zero = jnp.zeros((R, NH), jnp.int32)
    ke0, ke1 = _tf(_i32(_KG0), _i32(_KG1), zero, elem)
    ka0, ka1 = _tf(ke0, ke1, zero, zero)
    kb0, kb1 = _tf(ke0, ke1, zero, jnp.ones((R, NH), jnp.int32))
    u_boost = _uniform01(kb0, kb1)

    one_c = jnp.full((R, NH), 1, jnp.int32)
    two_c = jnp.full((R, NH), 2, jnp.int32)

    def inner_cond(st2):
        return jnp.any(st2[3] <= jnp.float32(0.0))

    def inner_body(st2):
        i0, i1, x, v = st2
        iact = v <= jnp.float32(0.0)
        i0, i1 = _tf(i0, i1, zero, zero)
        is0, is1 = _tf(i0, i1, zero, one_c)
        xn = _normal(is0, is1)
        vn = jnp.float32(1.0) + xn * c
        x = jnp.where(iact, xn, x)
        v = jnp.where(iact, vn, v)
        return (i0, i1, x, v)

    def draw(k0, k1, act):
        kx0, kx1 = _tf(k0, k1, zero, one_c)
        ku0, ku1 = _tf(k0, k1, zero, two_c)
        sub0, sub1 = _tf(kx0, kx1, zero, one_c)
        x = _normal(sub0, sub1)
        v = jnp.float32(1.0) + x * c
        if act is not None:
            v = jnp.where(act, v, jnp.float32(1.0))
        _, _, x, v = lax.while_loop(inner_cond, inner_body, (kx0, kx1, x, v))
        return x * x, (v * v) * v, _uniform01(ku0, ku1)

    X, V, U = draw(ka0, ka1, None)
    act32 = _accept_continue(X, V, U, d).astype(jnp.int32)

    def outer_cond(st):
        return jnp.any(st[5] != 0)

    def outer_body(st):
        k0, k1, X, V, U, act32 = st
        act = act32 != 0
        k0, k1 = _tf(k0, k1, zero, zero)
        Xn, Vn, Un = draw(k0, k1, act)
        X = jnp.where(act, Xn, X)
        V = jnp.where(act, Vn, V)
        U = jnp.where(act, Un, U)
        act32 = act32 & _accept_continue(X, V, U, d).astype(jnp.int32)
        return (k0, k1, X, V, U, act32)

    _, _, _, V, _, _ = lax.while_loop(
        outer_cond, outer_body, (ka0, ka1, X, V, U, act32))

    inv_a = jnp.float32(1.0) / a
    boost = jnp.where(mask_ge1, jnp.float32(1.0),
                      jnp.power(jnp.float32(1.0) - u_boost, inv_a))
    g = (d * V) * boost
    t = jnp.power(g, a)
    return x_mu + (x_sig * sgn) * t


@jax.jit
def kernel(latent_concat, W_mu, b_mu, W_sig, b_sig, W_p, b_p):
    nkey = jax.random.key(1234)
    kc = jax.random.randint(jax.random.fold_in(nkey, 1), (B, 1), 0, K)
    kc = kc.astype(jnp.int32)
    u_sign = jax.random.uniform(jax.random.fold_in(nkey, 3), (B, N, K))
    sgn_full = jnp.where(u_sign < 0.5, -1.0, 1.0).astype(jnp.int8)
    sgn = jnp.take_along_axis(sgn_full, kc.reshape(B, 1, 1), axis=2)
    sgn = sgn.reshape(B, N)

    grid = (B // R,)
    out = pl.pallas_call(
        _sampler_body,
        grid=grid,
        in_specs=[
            pl.BlockSpec((R, D), lambda i: (i, 0)),
            pl.BlockSpec((D, KN), lambda i: (0, 0)),
            pl.BlockSpec((1, KN), lambda i: (0, 0)),
            pl.BlockSpec((D, KN), lambda i: (0, 0)),
            pl.BlockSpec((1, KN), lambda i: (0, 0)),
            pl.BlockSpec((D, KN), lambda i: (0, 0)),
            pl.BlockSpec((1, KN), lambda i: (0, 0)),
            pl.BlockSpec((R, N), lambda i: (i, 0)),
            pl.BlockSpec((R, 1), lambda i: (i, 0)),
        ],
        out_specs=[
            pl.BlockSpec((R, N), lambda i: (i, 0)),
            pl.BlockSpec((R, KN), lambda i: (i, 0)),
            pl.BlockSpec((R, KN), lambda i: (i, 0)),
            pl.BlockSpec((R, KN), lambda i: (i, 0)),
        ],
        out_shape=[
            jax.ShapeDtypeStruct((B, N), jnp.float32),
            jax.ShapeDtypeStruct((B, KN), jnp.float32),
            jax.ShapeDtypeStruct((B, KN), jnp.float32),
            jax.ShapeDtypeStruct((B, KN), jnp.float32),
        ],
        compiler_params=pltpu.CompilerParams(
            dimension_semantics=("parallel",)),
    )(latent_concat, W_mu, b_mu.reshape(1, KN), W_sig, b_sig.reshape(1, KN),
      W_p, b_p.reshape(1, KN), sgn, kc)
    samples_x, x_mu, x_sig, x_p = out
    return (samples_x, x_mu.reshape(B, N, K), x_sig.reshape(B, N, K),
            x_p.reshape(B, N, K))


# speedup vs baseline: 1.1335x; 1.1335x over previous
import numpy as np
import jax
import jax.numpy as jnp
from jax import lax
from jax.experimental import pallas as pl
from jax.experimental.pallas import tpu as pltpu

B = 16384
D = 768
K = 4
N = 256
KN = K * N
R = 128
NH = 128

_U32 = np.uint32


def _np_tf2x32(k0, k1, x0, x1):
    ks2 = _U32(k0) ^ _U32(k1) ^ _U32(0x1BD11BDA)
    x0 = _U32(x0 + k0)
    x1 = _U32(x1 + k1)
    keys = [(k1, ks2, 1), (ks2, k0, 2), (k0, k1, 3), (k1, ks2, 4), (ks2, k0, 5)]
    rots = [(13, 15, 26, 6), (17, 29, 16, 24), (13, 15, 26, 6),
            (17, 29, 16, 24), (13, 15, 26, 6)]
    for (ka, kb, i), rr in zip(keys, rots):
        for r in rr:
            x0 = _U32((int(x0) + int(x1)) & 0xFFFFFFFF)
            x1 = _U32(((int(x1) << r) | (int(x1) >> (32 - r))) & 0xFFFFFFFF)
            x1 = x1 ^ x0
        x0 = _U32((int(x0) + int(ka)) & 0xFFFFFFFF)
        x1 = _U32((int(x1) + int(kb) + i) & 0xFFFFFFFF)
    return int(x0), int(x1)


_KG0, _KG1 = _np_tf2x32(0, 1234, 0, 2)


def _i32(x):
    return jnp.int32(np.int32(np.uint32(x)))


def _rotl(v, r):
    return lax.shift_left(v, jnp.int32(r)) | lax.shift_right_logical(
        v, jnp.int32(32 - r))


def _tf(k0, k1, x0, x1):
    ks2 = k0 ^ k1 ^ _i32(0x1BD11BDA)
    x0 = x0 + k0
    x1 = x1 + k1
    keys = [(k1, ks2, 1), (ks2, k0, 2), (k0, k1, 3), (k1, ks2, 4), (ks2, k0, 5)]
    rots = [(13, 15, 26, 6), (17, 29, 16, 24), (13, 15, 26, 6),
            (17, 29, 16, 24), (13, 15, 26, 6)]
    for (ka, kb, i), rr in zip(keys, rots):
        for r in rr:
            x0 = x0 + x1
            x1 = _rotl(x1, r)
            x1 = x1 ^ x0
        x0 = x0 + ka
        x1 = x1 + kb + jnp.int32(i)
    return x0, x1


def _bits_to_unit(bits):
    f = lax.bitcast_convert_type(
        lax.shift_right_logical(bits, jnp.int32(9)) | _i32(0x3F800000),
        jnp.float32)
    return f - jnp.float32(1.0)


def _uniform01(k0, k1):
    o0, o1 = _tf(k0, k1, jnp.zeros_like(k0), jnp.zeros_like(k1))
    return jnp.maximum(jnp.float32(0.0), _bits_to_unit(o0 ^ o1))


def _erf_inv(x):
    w = -jnp.log1p(-x * x)
    lt = w < jnp.float32(5.0)
    w1 = w - jnp.float32(2.5)
    p = jnp.float32(2.81022636e-08)
    for c in (3.43273939e-07, -3.5233877e-06, -4.39150654e-06, 0.00021858087,
              -0.00125372503, -0.00417768164, 0.246640727, 1.50140941):
        p = jnp.float32(c) + p * w1
    w2 = jnp.sqrt(w) - jnp.float32(3.0)
    q = jnp.float32(-0.000200214257)
    for c in (0.000100950558, 0.00134934322, -0.00367342844, 0.00573950773,
              -0.0076224613, 0.00943887047, 1.00167406, 2.83297682):
        q = jnp.float32(c) + q * w2
    return jnp.where(lt, p, q) * x


def _normal(k0, k1):
    lo = jnp.float32(-0.99999994)
    hi = jnp.float32(1.0)
    o0, o1 = _tf(k0, k1, jnp.zeros_like(k0), jnp.zeros_like(k1))
    u = _bits_to_unit(o0 ^ o1)
    u = jnp.maximum(lo, u * (hi - lo) + lo)
    return jnp.float32(1.41421354) * _erf_inv(u)


def _softplus(x):
    return jnp.maximum(x, jnp.float32(0.0)) + jnp.log1p(jnp.exp(-jnp.abs(x)))


def _accept_continue(X, V, U, d):
    sq = jnp.float32(1.0) - jnp.float32(0.0331) * (X * X)
    rhs = jnp.float32(0.5) * X + d * ((jnp.float32(1.0) - V) + jnp.log(V))
    return (U >= sq) & (jnp.log(U) >= rhs)


def _sampler_body(lat_ref, wmu_ref, bmu_ref, wsig_ref, bsig_ref, wp_ref,
                  bp_ref, sgn_ref, kc_ref, sx_ref, xmu_ref, xsig_ref, xp_ref):
    lat = lat_ref[...].astype(jnp.bfloat16)
    hi = jax.lax.Precision.HIGHEST
    x_mu = jnp.dot(lat, wmu_ref[...].astype(jnp.bfloat16),
                   preferred_element_type=jnp.float32) + bmu_ref[...]
    xs_l = jnp.dot(lat, wsig_ref[...].astype(jnp.bfloat16),
                   preferred_element_type=jnp.float32) + bsig_ref[...]
    xp_l = jnp.dot(lat, wp_ref[...].astype(jnp.bfloat16),
                   preferred_element_type=jnp.float32) + bp_ref[...]

    x_sig = _softplus(xs_l) + jnp.float32(1e-08)
    x_sig = jnp.where(x_sig > jnp.float32(4.0), jnp.float32(4.0), x_sig)
    x_sig = jnp.where(x_sig < jnp.float32(0.001), jnp.float32(0.001), x_sig)
    x_p = _softplus(xp_l) + jnp.float32(1e-08) + jnp.float32(0.1)
    x_p = jnp.where(x_p > jnp.float32(10.0), jnp.float32(10.0), x_p)

    xmu_ref[...] = x_mu
    xsig_ref[...] = x_sig
    xp_ref[...] = x_p

    kc = kc_ref[...]
    cols = lax.broadcasted_iota(jnp.int32, (R, KN), 1)
    mask = (cols % K == kc).astype(jnp.float32)
    s_rows = lax.broadcasted_iota(jnp.int32, (KN, N), 0)
    s_cols = lax.broadcasted_iota(jnp.int32, (KN, N), 1)
    S = (s_rows // K == s_cols).astype(jnp.float32)
    x_mu = jnp.dot(x_mu * mask, S, precision=hi,
                   preferred_element_type=jnp.float32)
    x_sig = jnp.dot(x_sig * mask, S, precision=hi,
                    preferred_element_type=jnp.float32)
    x_p = jnp.dot(x_p * mask, S, precision=hi,
                  preferred_element_type=jnp.float32)

    sgn_all = sgn_ref[...].astype(jnp.float32)
    for h in range(N // NH):
        cs = slice(h * NH, (h + 1) * NH)
        sx_ref[:, cs] = _sample_half(x_mu[:, cs], x_sig[:, cs], x_p[:, cs],
                                     sgn_all[:, cs], kc, h)


def _sample_half(x_mu, x_sig, x_p, sgn, kc, h):
    a = jnp.float32(1.0) / x_p
    mask_ge1 = a >= jnp.float32(1.0)
    alpha = jnp.where(mask_ge1, a, a + jnp.float32(1.0))
    third = jnp.float32(np.float32(1.0 / 3.0))
    d = alpha - third
    c = third / jnp.sqrt(d)

    rows = lax.broadcasted_iota(jnp.int32, (R, NH), 0)
    cols_n = lax.broadcasted_iota(jnp.int32, (R, NH), 1) + h * NH
    elem = (pl.program_id(0) * R + rows) * KN + cols_n * K + kc
    zero = jnp.zeros((R, NH), jnp.int32)
    ke0, ke1 = _tf(_i32(_KG0), _i32(_KG1), zero, elem)
    ka0, ka1 = _tf(ke0, ke1, zero, zero)
    kb0, kb1 = _tf(ke0, ke1, zero, jnp.ones((R, NH), jnp.int32))
    u_boost = _uniform01(kb0, kb1)

    one_c = jnp.full((R, NH), 1, jnp.int32)
    two_c = jnp.full((R, NH), 2, jnp.int32)

    def inner_cond(st2):
        return jnp.any(st2[3] <= jnp.float32(0.0))

    def inner_body(st2):
        i0, i1, x, v = st2
        iact = v <= jnp.float32(0.0)
        i0, i1 = _tf(i0, i1, zero, zero)
        is0, is1 = _tf(i0, i1, zero, one_c)
        xn = _normal(is0, is1)
        vn = jnp.float32(1.0) + xn * c
        x = jnp.where(iact, xn, x)
        v = jnp.where(iact, vn, v)
        return (i0, i1, x, v)

    def draw(k0, k1, act):
        kx0, kx1 = _tf(k0, k1, zero, one_c)
        ku0, ku1 = _tf(k0, k1, zero, two_c)
        sub0, sub1 = _tf(kx0, kx1, zero, one_c)
        x = _normal(sub0, sub1)
        v = jnp.float32(1.0) + x * c
        if act is not None:
            v = jnp.where(act, v, jnp.float32(1.0))
        _, _, x, v = lax.while_loop(inner_cond, inner_body, (kx0, kx1, x, v))
        return x * x, (v * v) * v, _uniform01(ku0, ku1)

    X, V, U = draw(ka0, ka1, None)
    act32 = _accept_continue(X, V, U, d).astype(jnp.int32)

    def outer_cond(st):
        return jnp.any(st[5] != 0)

    def outer_body(st):
        k0, k1, X, V, U, act32 = st
        act = act32 != 0
        k0, k1 = _tf(k0, k1, zero, zero)
        Xn, Vn, Un = draw(k0, k1, act)
        X = jnp.where(act, Xn, X)
        V = jnp.where(act, Vn, V)
        U = jnp.where(act, Un, U)
        act32 = act32 & _accept_continue(X, V, U, d).astype(jnp.int32)
        return (k0, k1, X, V, U, act32)

    _, _, _, V, _, _ = lax.while_loop(
        outer_cond, outer_body, (ka0, ka1, X, V, U, act32))

    inv_a = jnp.float32(1.0) / a
    boost = jnp.where(mask_ge1, jnp.float32(1.0),
                      jnp.power(jnp.float32(1.0) - u_boost, inv_a))
    g = (d * V) * boost
    t = jnp.power(g, a)
    return x_mu + (x_sig * sgn) * t


@jax.jit
def kernel(latent_concat, W_mu, b_mu, W_sig, b_sig, W_p, b_p):
    nkey = jax.random.key(1234)
    kc = jax.random.randint(jax.random.fold_in(nkey, 1), (B, 1), 0, K)
    kc = kc.astype(jnp.int32)
    u_sign = jax.random.uniform(jax.random.fold_in(nkey, 3), (B, N, K))
    sgn_full = jnp.where(u_sign < 0.5, -1.0, 1.0).astype(jnp.int8)
    sgn = jnp.take_along_axis(sgn_full, kc.reshape(B, 1, 1), axis=2)
    sgn = sgn.reshape(B, N)

    grid = (B // R,)
    out = pl.pallas_call(
        _sampler_body,
        grid=grid,
        in_specs=[
            pl.BlockSpec((R, D), lambda i: (i, 0)),
            pl.BlockSpec((D, KN), lambda i: (0, 0)),
            pl.BlockSpec((1, KN), lambda i: (0, 0)),
            pl.BlockSpec((D, KN), lambda i: (0, 0)),
            pl.BlockSpec((1, KN), lambda i: (0, 0)),
            pl.BlockSpec((D, KN), lambda i: (0, 0)),
            pl.BlockSpec((1, KN), lambda i: (0, 0)),
            pl.BlockSpec((R, N), lambda i: (i, 0)),
            pl.BlockSpec((R, 1), lambda i: (i, 0)),
        ],
        out_specs=[
            pl.BlockSpec((R, N), lambda i: (i, 0)),
            pl.BlockSpec((R, KN), lambda i: (i, 0)),
            pl.BlockSpec((R, KN), lambda i: (i, 0)),
            pl.BlockSpec((R, KN), lambda i: (i, 0)),
        ],
        out_shape=[
            jax.ShapeDtypeStruct((B, N), jnp.float32),
            jax.ShapeDtypeStruct((B, KN), jnp.float32),
            jax.ShapeDtypeStruct((B, KN), jnp.float32),
            jax.ShapeDtypeStruct((B, KN), jnp.float32),
        ],
        compiler_params=pltpu.CompilerParams(
            dimension_semantics=("parallel",)),
    )(latent_concat, W_mu, b_mu.reshape(1, KN), W_sig, b_sig.reshape(1, KN),
      W_p, b_p.reshape(1, KN), sgn, kc)
    samples_x, x_mu, x_sig, x_p = out
    return (samples_x, x_mu.reshape(B, N, K), x_sig.reshape(B, N, K),
            x_p.reshape(B, N, K))
